# baseline (speedup 1.0000x reference)
import jax
import jax.numpy as jnp
from jax import lax
from jax.experimental import pallas as pl
from jax.experimental.pallas import tpu as pltpu
from jax.experimental.pallas import tpu_sc as plsc

B = 16384
D = 128
T = 100000
NC, NS, L = 2, 16, 16
NW = NC * NS
N_BIG = 20
R_BIG, R_SMALL = 3128, 3120
WPOS_PAD = 3136
COMP_PAD = 3200
C = 128
MOM = 0.1


def _body(feat_hbm, idx_hbm, lab_hbm, mem_hbm, cls_hbm,
          out_feat, out_cls,
          idx_v, lab_v, wpos, comp_pos, comp_loc, glob2d, cls_v,
          fbuf, obuf, rot, sem_cp, sem_g0, sem_g1, sem_sc):
    w = lax.axis_index("s") * NC + lax.axis_index("c")
    big = w < N_BIG
    base = jnp.where(big, w * R_BIG, N_BIG * R_BIG + (w - N_BIG) * R_SMALL)
    nrows = jnp.where(big, R_BIG, R_SMALL)
    iota = lax.iota(jnp.int32, L)

    @pl.when(big)
    def _():
        pltpu.async_copy(mem_hbm.at[pl.ds(base, R_BIG)],
                         out_feat.at[pl.ds(base, R_BIG)], sem_cp)
        pltpu.sync_copy(cls_hbm.at[pl.ds(base, R_BIG)],
                        cls_v.at[pl.ds(0, R_BIG)])

    @pl.when(jnp.logical_not(big))
    def _():
        pltpu.async_copy(mem_hbm.at[pl.ds(base, R_SMALL)],
                         out_feat.at[pl.ds(base, R_SMALL)], sem_cp)
        pltpu.sync_copy(cls_hbm.at[pl.ds(base, R_SMALL)],
                        cls_v.at[pl.ds(0, R_SMALL)])

    pltpu.sync_copy(idx_hbm, idx_v)
    pltpu.sync_copy(lab_hbm, lab_v)

    def init_body(i, _):
        wpos[pl.ds(i * L, L)] = jnp.full((L,), -1, jnp.int32)
        return 0
    lax.fori_loop(0, WPOS_PAD // L, init_body, 0)

    def scan_body(v, _):
        x = idx_v[pl.ds(v * L, L)]
        rot[pl.ds(0, L)] = x
        rot[pl.ds(L, L)] = x
        loc = x - base
        m_in = (loc >= 0) & (loc < nrows)
        dup = jnp.zeros((L,), jnp.bool_)
        for s in range(1, L):
            xs = rot[pl.ds(s, L)]
            dup = dup | ((x == xs) & (iota < (L - s)))
        m_fin = m_in & jnp.logical_not(dup)
        posv = jnp.full((L,), v * L, jnp.int32) + iota
        plsc.store_scatter(wpos, [loc], posv, mask=m_fin)
        return 0
    lax.fori_loop(0, B // L, scan_body, 0)

    def comp_body(v, mt):
        wp = wpos[pl.ds(v * L, L)]
        m = wp >= jnp.zeros((L,), jnp.int32)
        cnt = jnp.sum(jnp.where(m, 1, 0).astype(jnp.int32))
        plsc.store_compressed(comp_pos.at[pl.ds(mt, L)], wp, mask=m)
        locs = jnp.full((L,), v * L, jnp.int32) + iota
        plsc.store_compressed(comp_loc.at[pl.ds(mt, L)], locs, mask=m)
        return mt + cnt
    M = lax.fori_loop(0, WPOS_PAD // L, comp_body, jnp.int32(0))
    Mpad = ((M + C - 1) // C) * C

    @pl.when(M > 0)
    def _():
        pv = jnp.full((L,), comp_pos[pl.ds(0, L)][0], jnp.int32)
        lv = jnp.full((L,), comp_loc[pl.ds(0, L)][0], jnp.int32)
        def pad_body(t, _):
            lanes = jnp.full((L,), t * L, jnp.int32) + iota
            mfill = lanes >= M
            plsc.store_scatter(comp_pos, [lanes], pv, mask=mfill)
            plsc.store_scatter(comp_loc, [lanes], lv, mask=mfill)
            return 0
        lax.fori_loop(M // L, Mpad // L, pad_body, 0)

    def cls_body(t, _):
        pos16 = comp_pos[pl.ds(t * L, L)]
        labs = plsc.load_gather(lab_v, [pos16])
        rows16 = comp_loc[pl.ds(t * L, L)]
        plsc.store_scatter(cls_v, [rows16], labs)
        return 0
    lax.fori_loop(0, Mpad // L, cls_body, 0)

    @pl.when(big)
    def _():
        pltpu.sync_copy(cls_v.at[pl.ds(0, R_BIG)],
                        out_cls.at[pl.ds(base, R_BIG)])
        pltpu.make_async_copy(mem_hbm.at[pl.ds(base, R_BIG)],
                              out_feat.at[pl.ds(base, R_BIG)], sem_cp).wait()

    @pl.when(jnp.logical_not(big))
    def _():
        pltpu.sync_copy(cls_v.at[pl.ds(0, R_SMALL)],
                        out_cls.at[pl.ds(base, R_SMALL)])
        pltpu.make_async_copy(mem_hbm.at[pl.ds(base, R_SMALL)],
                              out_feat.at[pl.ds(base, R_SMALL)], sem_cp).wait()

    def rp_body(r, _):
        g = comp_loc[pl.ds(r * L, L)] + base
        glob2d[r // 8, pl.ds((r % 8) * L, L)] = g
        return 0
    lax.fori_loop(0, Mpad // L, rp_body, 0)

    def ch_body(c, _):
        cpa = pltpu.async_copy(feat_hbm.at[comp_pos.at[pl.ds(c * C, C)]],
                               fbuf, sem_g0)
        cpb = pltpu.async_copy(mem_hbm.at[glob2d.at[c]], obuf, sem_g1)
        cpa.wait()
        cpb.wait()
        def fm(t, _):
            i = t // 8
            jo = (t % 8) * L
            obuf[i, pl.ds(jo, L)] = (obuf[i, pl.ds(jo, L)] * (1.0 - MOM)
                                     + fbuf[i, pl.ds(jo, L)] * MOM)
            return 0
        lax.fori_loop(0, C * (D // L), fm, 0)
        pltpu.async_copy(obuf, out_feat.at[glob2d.at[c]], sem_sc).wait()
        return 0
    lax.fori_loop(0, Mpad // C, ch_body, 0)


def kernel(feature, index_target, label_target,
           target_featurememory, target_classmemory):
    k = pl.kernel(
        _body,
        out_type=(jax.ShapeDtypeStruct((T, D), jnp.float32),
                  jax.ShapeDtypeStruct((T,), jnp.int32)),
        mesh=plsc.VectorSubcoreMesh(core_axis_name="c", subcore_axis_name="s"),
        compiler_params=pltpu.CompilerParams(needs_layout_passes=False),
        scratch_types=[
            pltpu.VMEM((B,), jnp.int32),
            pltpu.VMEM((B,), jnp.int32),
            pltpu.VMEM((WPOS_PAD,), jnp.int32),
            pltpu.VMEM((COMP_PAD,), jnp.int32),
            pltpu.VMEM((COMP_PAD,), jnp.int32),
            pltpu.VMEM((COMP_PAD // C, C), jnp.int32),
            pltpu.VMEM((WPOS_PAD,), jnp.int32),
            pltpu.VMEM((C, D), jnp.float32),
            pltpu.VMEM((C, D), jnp.float32),
            pltpu.VMEM((2 * L,), jnp.int32),
            pltpu.SemaphoreType.DMA,
            pltpu.SemaphoreType.DMA,
            pltpu.SemaphoreType.DMA,
            pltpu.SemaphoreType.DMA,
        ],
    )
    return k(feature, index_target, label_target,
             target_featurememory, target_classmemory)
